# baseline (speedup 1.0000x reference)
import functools

import jax
import jax.numpy as jnp
from jax import lax
from jax.experimental import pallas as pl
from jax.experimental.pallas import tpu as pltpu
from jax.experimental.pallas import tpu_sc as plsc

NC = 2
NS = 16
NW = NC * NS
CHUNK = 128
NBUF = 8
NRB = 2
LANES = 16


def _tc_in_proj(x, wcat):
    n, d_in = x.shape
    bn = 1000

    def body(x_ref, w_ref, y1_ref, xr_ref):
        z = lax.dot_general(x_ref[...], w_ref[...], (((1,), (1,)), ((), ())),
                            preferred_element_type=jnp.float32)
        y1_ref[...] = z[:, :64]
        xr_ref[...] = z[:, 64:]

    return pl.pallas_call(
        body,
        grid=(n // bn,),
        in_specs=[pl.BlockSpec((bn, d_in), lambda i: (i, 0)),
                  pl.BlockSpec((d_in, d_in), lambda i: (0, 0))],
        out_specs=[pl.BlockSpec((bn, 64), lambda i: (i, 0)),
                   pl.BlockSpec((bn, 64), lambda i: (i, 0))],
        out_shape=[jax.ShapeDtypeStruct((n, 64), jnp.float32),
                   jax.ShapeDtypeStruct((n, 64), jnp.float32)],
    )(x, wcat)


def _sc_aggregate(tab, src2d, dst2d, with_count, rpw0):
    n, d = tab.shape
    rtot = src2d.shape[0]
    rpw1 = rtot // NS - rpw0
    npad = ((n + 1 + NS * CHUNK - 1) // (NS * CHUNK)) * (NS * CHUNK)
    stripe = npad // NS
    zrows = npad // NS

    mesh = plsc.VectorSubcoreMesh(core_axis_name="c", subcore_axis_name="s")

    out_type = [jax.ShapeDtypeStruct((NC, npad, d), jnp.float32)]
    scratch = (
        [pltpu.VMEM((NBUF, CHUNK), jnp.int32),
         pltpu.VMEM((NBUF, CHUNK), jnp.int32)]
        + [pltpu.VMEM((CHUNK, d), jnp.float32) for _ in range(NRB)]
        + [pltpu.VMEM((CHUNK, LANES), jnp.float32),
           pltpu.VMEM_SHARED((npad, d), jnp.float32)]
        + [pltpu.SemaphoreType.DMA for _ in range(NRB)]
    )
    if with_count:
        out_type.append(jax.ShapeDtypeStruct((NC, npad, LANES), jnp.float32))
        scratch.append(pltpu.VMEM_SHARED((npad, LANES), jnp.float32))

    @functools.partial(
        pl.kernel, out_type=out_type, mesh=mesh, scratch_types=scratch,
        compiler_params=pltpu.CompilerParams(use_tc_tiling_on_sc=False))
    def k(*refs):
        nout = 2 if with_count else 1
        tab_hbm, src_hbm, dst_hbm = refs[:3]
        s_hbm = refs[3]
        cnt_hbm = refs[4] if with_count else None
        r0 = 3 + nout
        isrc, idst = refs[r0], refs[r0 + 1]
        rbufs = refs[r0 + 2:r0 + 2 + NRB]
        ones = refs[r0 + 2 + NRB]
        acc = refs[r0 + 3 + NRB]
        gsems = refs[r0 + 4 + NRB:r0 + 4 + 2 * NRB]
        cacc = refs[r0 + 4 + 2 * NRB] if with_count else None
        c = lax.axis_index("c")
        s = lax.axis_index("s")
        rpw_me = jnp.where(c == 0, rpw0, rpw1)
        base = jnp.where(c == 0, s * rpw0, NS * rpw0 + s * rpw1)

        zbuf = rbufs[0]

        @pl.loop(0, CHUNK)
        def _(r):
            ones[r, pl.ds(0, LANES)] = jnp.zeros((LANES,), jnp.float32)

            @pl.loop(0, d, step=LANES)
            def _(cc):
                zbuf[r, pl.ds(cc, LANES)] = jnp.zeros((LANES,), jnp.float32)

        @pl.loop(0, zrows, step=CHUNK)
        def _(r):
            pltpu.sync_copy(zbuf, acc.at[pl.ds(s * zrows + r, CHUNK)])
            if with_count:
                pltpu.sync_copy(ones, cacc.at[pl.ds(s * zrows + r, CHUNK)])

        if with_count:
            @pl.loop(0, CHUNK)
            def _(r):
                ones[r, pl.ds(0, LANES)] = jnp.ones((LANES,), jnp.float32)

        plsc.subcore_barrier()

        @pl.loop(0, rpw_me, step=NBUF)
        def _(m):
            r = base + m
            pltpu.sync_copy(src_hbm.at[pl.ds(r, NBUF)], isrc)
            pltpu.sync_copy(dst_hbm.at[pl.ds(r, NBUF)], idst)
            cps = [None, None]
            cps[0] = pltpu.async_copy(tab_hbm.at[isrc.at[0]], rbufs[0],
                                      gsems[0])
            for b in range(NBUF):
                cur = b % 2
                cps[cur].wait()
                if b + 1 < NBUF:
                    cps[1 - cur] = pltpu.async_copy(
                        tab_hbm.at[isrc.at[b + 1]], rbufs[1 - cur],
                        gsems[1 - cur])
                if with_count:
                    pltpu.sync_copy(ones, cacc.at[idst.at[b]], add=True)
                pltpu.sync_copy(rbufs[cur], acc.at[idst.at[b]], add=True)

        plsc.subcore_barrier()

        pltpu.sync_copy(acc.at[pl.ds(s * stripe, stripe)],
                        s_hbm.at[c, pl.ds(s * stripe, stripe)])
        if with_count:
            pltpu.sync_copy(cacc.at[pl.ds(s * stripe, stripe)],
                            cnt_hbm.at[c, pl.ds(s * stripe, stripe)])

    return k(tab, src2d, dst2d)


def _tc_mid(s1, cnt, xr, b1, w2cat):
    n = xr.shape[0]
    bn = 1000

    def body(s1_ref, cnt_ref, xr_ref, b1_ref, w_ref, z2_ref, hr_ref):
        sb = s1_ref[...]
        cb = cnt_ref[...]
        ssum = sb[0] + sb[1]
        csum = cb[0, :, 0:1] + cb[1, :, 0:1]
        mean = ssum / jnp.maximum(csum, 1.0)
        h = jnp.maximum(mean + xr_ref[...] + b1_ref[...], 0.0)
        z = lax.dot_general(h, w_ref[...], (((1,), (1,)), ((), ())),
                            preferred_element_type=jnp.float32)
        z2_ref[...] = z[:, :LANES]
        hr_ref[...] = z[:, LANES:LANES + 2]

    return pl.pallas_call(
        body,
        grid=(n // bn,),
        in_specs=[pl.BlockSpec((NC, bn, 64), lambda i: (0, i, 0)),
                  pl.BlockSpec((NC, bn, LANES), lambda i: (0, i, 0)),
                  pl.BlockSpec((bn, 64), lambda i: (i, 0)),
                  pl.BlockSpec((1, 64), lambda i: (0, 0)),
                  pl.BlockSpec((2 * LANES, 64), lambda i: (0, 0))],
        out_specs=[pl.BlockSpec((bn, LANES), lambda i: (i, 0)),
                   pl.BlockSpec((bn, 2), lambda i: (i, 0))],
        out_shape=[jax.ShapeDtypeStruct((n, LANES), jnp.float32),
                   jax.ShapeDtypeStruct((n, 2), jnp.float32)],
    )(s1, cnt, xr, b1, w2cat)


def _tc_out(s2, cnt, hr, b2):
    n = hr.shape[0]
    bn = 1000

    def body(s2_ref, cnt_ref, hr_ref, b2_ref, o_ref):
        sb = s2_ref[...]
        cb = cnt_ref[...]
        ssum = sb[0] + sb[1]
        csum = cb[0, :, 0:1] + cb[1, :, 0:1]
        v = ssum[:, 0:2] / jnp.maximum(csum, 1.0) + hr_ref[...] + b2_ref[...]
        m = jnp.max(v, axis=1, keepdims=True)
        lse = jnp.log(jnp.sum(jnp.exp(v - m), axis=1, keepdims=True))
        o_ref[...] = v - m - lse

    return pl.pallas_call(
        body,
        grid=(n // bn,),
        in_specs=[pl.BlockSpec((NC, bn, LANES), lambda i: (0, i, 0)),
                  pl.BlockSpec((NC, bn, LANES), lambda i: (0, i, 0)),
                  pl.BlockSpec((bn, 2), lambda i: (i, 0)),
                  pl.BlockSpec((1, 2), lambda i: (0, 0))],
        out_specs=pl.BlockSpec((bn, 2), lambda i: (i, 0)),
        out_shape=jax.ShapeDtypeStruct((n, 2), jnp.float32),
    )(s2, cnt, hr, b2)


def kernel(x, edge_index, W1l, W1r, b1, W2l, W2r, b2):
    n = x.shape[0]
    e = edge_index.shape[1]

    src = edge_index[0].astype(jnp.int32)
    dst = edge_index[1].astype(jnp.int32)

    block = NW * CHUNK * NBUF
    e_pad = ((e + block - 1) // block) * block
    pad = e_pad - e
    src = jnp.concatenate([src, jnp.zeros((pad,), jnp.int32)])
    dst = jnp.concatenate([dst, jnp.full((pad,), n, jnp.int32)])
    src2d = src.reshape(e_pad // CHUNK, CHUNK)
    dst2d = dst.reshape(e_pad // CHUNK, CHUNK)

    wcat = jnp.concatenate([W1l, W1r], axis=0)
    w2cat = jnp.zeros((2 * LANES, 64), jnp.float32)
    w2cat = w2cat.at[0:2].set(W2l).at[LANES:LANES + 2].set(W2r)
    b1r = b1.reshape(1, 64)
    b2r = b2.reshape(1, 2)

    y1, xr = _tc_in_proj(x, wcat)
    s1, cnt = _sc_aggregate(y1, src2d, dst2d, with_count=True, rpw0=56)
    z2, hr = _tc_mid(s1, cnt, xr, b1r, w2cat)
    (s2,) = _sc_aggregate(z2, src2d, dst2d, with_count=False, rpw0=64)
    return _tc_out(s2, cnt, hr, b2r)


# speedup vs baseline: 1.1765x; 1.1765x over previous
import functools

import jax
import jax.numpy as jnp
from jax import lax
from jax.experimental import pallas as pl
from jax.experimental.pallas import tpu as pltpu
from jax.experimental.pallas import tpu_sc as plsc

NC = 2
NS = 16
NW = NC * NS
CHUNK = 128
NBUF = 8
NRB = 2
LANES = 16


def _tc_in_proj(x, wcat):
    n, d_in = x.shape
    bn = 1000

    def body(x_ref, w_ref, y1_ref, xr_ref):
        z = lax.dot_general(x_ref[...], w_ref[...], (((1,), (1,)), ((), ())),
                            preferred_element_type=jnp.float32)
        y1_ref[...] = z[:, :64]
        xr_ref[...] = z[:, 64:]

    return pl.pallas_call(
        body,
        grid=(n // bn,),
        in_specs=[pl.BlockSpec((bn, d_in), lambda i: (i, 0)),
                  pl.BlockSpec((d_in, d_in), lambda i: (0, 0))],
        out_specs=[pl.BlockSpec((bn, 64), lambda i: (i, 0)),
                   pl.BlockSpec((bn, 64), lambda i: (i, 0))],
        out_shape=[jax.ShapeDtypeStruct((n, 64), jnp.float32),
                   jax.ShapeDtypeStruct((n, 64), jnp.float32)],
    )(x, wcat)


def _sc_aggregate(tab, src2d, dst2d, with_count, rpw0):
    n, d = tab.shape
    rtot = src2d.shape[0]
    rpw1 = rtot // NS - rpw0
    npad = ((n + 1 + NS * CHUNK - 1) // (NS * CHUNK)) * (NS * CHUNK)
    stripe = npad // NS
    zrows = npad // NS

    mesh = plsc.VectorSubcoreMesh(core_axis_name="c", subcore_axis_name="s")

    out_type = [jax.ShapeDtypeStruct((NC, npad, d), jnp.float32)]
    scratch = (
        [pltpu.VMEM((NBUF, CHUNK), jnp.int32),
         pltpu.VMEM((NBUF, CHUNK), jnp.int32)]
        + [pltpu.VMEM((CHUNK, d), jnp.float32) for _ in range(NRB)]
        + [pltpu.VMEM((CHUNK, LANES), jnp.float32),
           pltpu.VMEM_SHARED((npad, d), jnp.float32)]
        + [pltpu.SemaphoreType.DMA for _ in range(NRB)]
    )
    if with_count:
        out_type.append(jax.ShapeDtypeStruct((NC, npad, LANES), jnp.float32))
        scratch.append(pltpu.VMEM_SHARED((npad, LANES), jnp.float32))

    @functools.partial(
        pl.kernel, out_type=out_type, mesh=mesh, scratch_types=scratch,
        compiler_params=pltpu.CompilerParams(use_tc_tiling_on_sc=False))
    def k(*refs):
        nout = 2 if with_count else 1
        tab_hbm, src_hbm, dst_hbm = refs[:3]
        s_hbm = refs[3]
        cnt_hbm = refs[4] if with_count else None
        r0 = 3 + nout
        isrc, idst = refs[r0], refs[r0 + 1]
        rbufs = refs[r0 + 2:r0 + 2 + NRB]
        ones = refs[r0 + 2 + NRB]
        acc = refs[r0 + 3 + NRB]
        gsems = refs[r0 + 4 + NRB:r0 + 4 + 2 * NRB]
        cacc = refs[r0 + 4 + 2 * NRB] if with_count else None
        c = lax.axis_index("c")
        s = lax.axis_index("s")
        rpw_me = jnp.where(c == 0, rpw0, rpw1)
        base = jnp.where(c == 0, s * rpw0, NS * rpw0 + s * rpw1)

        zbuf = rbufs[0]

        @pl.loop(0, CHUNK)
        def _(r):
            ones[r, pl.ds(0, LANES)] = jnp.zeros((LANES,), jnp.float32)

            @pl.loop(0, d, step=LANES)
            def _(cc):
                zbuf[r, pl.ds(cc, LANES)] = jnp.zeros((LANES,), jnp.float32)

        @pl.loop(0, zrows, step=CHUNK)
        def _(r):
            pltpu.sync_copy(zbuf, acc.at[pl.ds(s * zrows + r, CHUNK)])
            if with_count:
                pltpu.sync_copy(ones, cacc.at[pl.ds(s * zrows + r, CHUNK)])

        if with_count:
            @pl.loop(0, CHUNK)
            def _(r):
                ones[r, pl.ds(0, LANES)] = jnp.ones((LANES,), jnp.float32)

        plsc.subcore_barrier()

        @pl.loop(0, rpw_me, step=NBUF)
        def _(m):
            r = base + m
            pltpu.sync_copy(src_hbm.at[pl.ds(r, NBUF)], isrc)
            pltpu.sync_copy(dst_hbm.at[pl.ds(r, NBUF)], idst)
            cps = [None, None]
            cps[0] = pltpu.async_copy(tab_hbm.at[isrc.at[0]], rbufs[0],
                                      gsems[0])
            for b in range(NBUF):
                cur = b % 2
                cps[cur].wait()
                if b + 1 < NBUF:
                    cps[1 - cur] = pltpu.async_copy(
                        tab_hbm.at[isrc.at[b + 1]], rbufs[1 - cur],
                        gsems[1 - cur])
                if with_count:
                    pltpu.sync_copy(ones, cacc.at[idst.at[b]], add=True)
                pltpu.sync_copy(rbufs[cur], acc.at[idst.at[b]], add=True)

        plsc.subcore_barrier()

        pltpu.sync_copy(acc.at[pl.ds(s * stripe, stripe)],
                        s_hbm.at[c, pl.ds(s * stripe, stripe)])
        if with_count:
            pltpu.sync_copy(cacc.at[pl.ds(s * stripe, stripe)],
                            cnt_hbm.at[c, pl.ds(s * stripe, stripe)])

    return k(tab, src2d, dst2d)


def _tc_mid(s1, cnt, xr, b1, w2cat):
    n = xr.shape[0]
    bn = 1000

    def body(s1_ref, cnt_ref, xr_ref, b1_ref, w_ref, z2_ref, hr_ref):
        sb = s1_ref[...]
        cb = cnt_ref[...]
        ssum = sb[0] + sb[1]
        csum = cb[0, :, 0:1] + cb[1, :, 0:1]
        mean = ssum / jnp.maximum(csum, 1.0)
        h = jnp.maximum(mean + xr_ref[...] + b1_ref[...], 0.0)
        z = lax.dot_general(h, w_ref[...], (((1,), (1,)), ((), ())),
                            preferred_element_type=jnp.float32)
        z2_ref[...] = z[:, :LANES]
        hr_ref[...] = z[:, LANES:LANES + 2]

    return pl.pallas_call(
        body,
        grid=(n // bn,),
        in_specs=[pl.BlockSpec((NC, bn, 64), lambda i: (0, i, 0)),
                  pl.BlockSpec((NC, bn, LANES), lambda i: (0, i, 0)),
                  pl.BlockSpec((bn, 64), lambda i: (i, 0)),
                  pl.BlockSpec((1, 64), lambda i: (0, 0)),
                  pl.BlockSpec((2 * LANES, 64), lambda i: (0, 0))],
        out_specs=[pl.BlockSpec((bn, LANES), lambda i: (i, 0)),
                   pl.BlockSpec((bn, 2), lambda i: (i, 0))],
        out_shape=[jax.ShapeDtypeStruct((n, LANES), jnp.float32),
                   jax.ShapeDtypeStruct((n, 2), jnp.float32)],
    )(s1, cnt, xr, b1, w2cat)


def _tc_out(s2, cnt, hr, b2):
    n = hr.shape[0]
    bn = 1000

    def body(s2_ref, cnt_ref, hr_ref, b2_ref, o_ref):
        sb = s2_ref[...]
        cb = cnt_ref[...]
        ssum = sb[0] + sb[1]
        csum = cb[0, :, 0:1] + cb[1, :, 0:1]
        v = ssum[:, 0:2] / jnp.maximum(csum, 1.0) + hr_ref[...] + b2_ref[...]
        m = jnp.max(v, axis=1, keepdims=True)
        lse = jnp.log(jnp.sum(jnp.exp(v - m), axis=1, keepdims=True))
        o_ref[...] = v - m - lse

    return pl.pallas_call(
        body,
        grid=(n // bn,),
        in_specs=[pl.BlockSpec((NC, bn, LANES), lambda i: (0, i, 0)),
                  pl.BlockSpec((NC, bn, LANES), lambda i: (0, i, 0)),
                  pl.BlockSpec((bn, 2), lambda i: (i, 0)),
                  pl.BlockSpec((1, 2), lambda i: (0, 0))],
        out_specs=pl.BlockSpec((bn, 2), lambda i: (i, 0)),
        out_shape=jax.ShapeDtypeStruct((n, 2), jnp.float32),
    )(s2, cnt, hr, b2)


def kernel(x, edge_index, W1l, W1r, b1, W2l, W2r, b2):
    n = x.shape[0]
    e = edge_index.shape[1]

    src = edge_index[0].astype(jnp.int32)
    dst = edge_index[1].astype(jnp.int32)

    block = NW * CHUNK * NBUF
    e_pad = ((e + block - 1) // block) * block
    pad = e_pad - e
    src = jnp.concatenate([src, jnp.zeros((pad,), jnp.int32)])
    dst = jnp.concatenate([dst, jnp.full((pad,), n, jnp.int32)])
    src2d = src.reshape(e_pad // CHUNK, CHUNK)
    dst2d = dst.reshape(e_pad // CHUNK, CHUNK)

    wcat = jnp.concatenate([W1l, W1r], axis=0)
    w2cat = jnp.zeros((2 * LANES, 64), jnp.float32)
    w2cat = w2cat.at[0:2].set(W2l).at[LANES:LANES + 2].set(W2r)
    b1r = b1.reshape(1, 64)
    b2r = b2.reshape(1, 2)

    y1, xr = _tc_in_proj(x, wcat)
    s1, cnt = _sc_aggregate(y1, src2d, dst2d, with_count=True, rpw0=104)
    z2, hr = _tc_mid(s1, cnt, xr, b1r, w2cat)
    (s2,) = _sc_aggregate(z2, src2d, dst2d, with_count=False, rpw0=96)
    return _tc_out(s2, cnt, hr, b2r)


# speedup vs baseline: 1.2008x; 1.0206x over previous
import functools

import jax
import jax.numpy as jnp
from jax import lax
from jax.experimental import pallas as pl
from jax.experimental.pallas import tpu as pltpu
from jax.experimental.pallas import tpu_sc as plsc

NC = 2
NS = 16
NW = NC * NS
CHUNK = 128
NBUF = 8
NRB = 2
LANES = 16


def _tc_in_proj(x, wcat):
    n, d_in = x.shape
    bn = 1000

    def body(x_ref, w_ref, y1_ref, xr_ref):
        z = lax.dot_general(x_ref[...], w_ref[...], (((1,), (1,)), ((), ())),
                            preferred_element_type=jnp.float32)
        y1_ref[...] = z[:, :64]
        xr_ref[...] = z[:, 64:]

    return pl.pallas_call(
        body,
        grid=(n // bn,),
        in_specs=[pl.BlockSpec((bn, d_in), lambda i: (i, 0)),
                  pl.BlockSpec((d_in, d_in), lambda i: (0, 0))],
        out_specs=[pl.BlockSpec((bn, 64), lambda i: (i, 0)),
                   pl.BlockSpec((bn, 64), lambda i: (i, 0))],
        out_shape=[jax.ShapeDtypeStruct((n, 64), jnp.float32),
                   jax.ShapeDtypeStruct((n, 64), jnp.float32)],
    )(x, wcat)


def _sc_aggregate(tab, src2d, dst2d, with_count, rpw0):
    n, d = tab.shape
    rtot = src2d.shape[0]
    rpw1 = rtot // NS - rpw0
    npad = ((n + 1 + NS * CHUNK - 1) // (NS * CHUNK)) * (NS * CHUNK)
    stripe = npad // NS
    zrows = npad // NS

    mesh = plsc.VectorSubcoreMesh(core_axis_name="c", subcore_axis_name="s")

    out_type = [jax.ShapeDtypeStruct((NC, npad, d), jnp.float32)]
    scratch = (
        [pltpu.VMEM((NBUF, CHUNK), jnp.int32),
         pltpu.VMEM((NBUF, CHUNK), jnp.int32)]
        + [pltpu.VMEM((CHUNK, d), jnp.float32) for _ in range(NRB)]
        + [pltpu.VMEM((CHUNK, LANES), jnp.float32),
           pltpu.VMEM_SHARED((npad, d), jnp.float32)]
        + [pltpu.SemaphoreType.DMA for _ in range(NRB)]
    )
    if with_count:
        out_type.append(jax.ShapeDtypeStruct((NC, npad, LANES), jnp.float32))
        scratch.append(pltpu.VMEM_SHARED((npad, LANES), jnp.float32))

    @functools.partial(
        pl.kernel, out_type=out_type, mesh=mesh, scratch_types=scratch,
        compiler_params=pltpu.CompilerParams(use_tc_tiling_on_sc=False))
    def k(*refs):
        nout = 2 if with_count else 1
        tab_hbm, src_hbm, dst_hbm = refs[:3]
        s_hbm = refs[3]
        cnt_hbm = refs[4] if with_count else None
        r0 = 3 + nout
        isrc, idst = refs[r0], refs[r0 + 1]
        rbufs = refs[r0 + 2:r0 + 2 + NRB]
        ones = refs[r0 + 2 + NRB]
        acc = refs[r0 + 3 + NRB]
        gsems = refs[r0 + 4 + NRB:r0 + 4 + 2 * NRB]
        cacc = refs[r0 + 4 + 2 * NRB] if with_count else None
        c = lax.axis_index("c")
        s = lax.axis_index("s")
        rpw_me = jnp.where(c == 0, rpw0, rpw1)
        base = jnp.where(c == 0, s * rpw0, NS * rpw0 + s * rpw1)

        zbuf = rbufs[0]

        @pl.loop(0, CHUNK)
        def _(r):
            ones[r, pl.ds(0, LANES)] = jnp.zeros((LANES,), jnp.float32)

            @pl.loop(0, d, step=LANES)
            def _(cc):
                zbuf[r, pl.ds(cc, LANES)] = jnp.zeros((LANES,), jnp.float32)

        @pl.loop(0, zrows, step=CHUNK)
        def _(r):
            pltpu.sync_copy(zbuf, acc.at[pl.ds(s * zrows + r, CHUNK)])
            if with_count:
                pltpu.sync_copy(ones, cacc.at[pl.ds(s * zrows + r, CHUNK)])

        if with_count:
            @pl.loop(0, CHUNK)
            def _(r):
                ones[r, pl.ds(0, LANES)] = jnp.ones((LANES,), jnp.float32)

        plsc.subcore_barrier()

        @pl.loop(0, rpw_me, step=NBUF)
        def _(m):
            r = base + m
            pltpu.sync_copy(src_hbm.at[pl.ds(r, NBUF)], isrc)
            pltpu.sync_copy(dst_hbm.at[pl.ds(r, NBUF)], idst)
            cps = [None, None]
            cps[0] = pltpu.async_copy(tab_hbm.at[isrc.at[0]], rbufs[0],
                                      gsems[0])
            for b in range(NBUF):
                cur = b % 2
                cps[cur].wait()
                if b + 1 < NBUF:
                    cps[1 - cur] = pltpu.async_copy(
                        tab_hbm.at[isrc.at[b + 1]], rbufs[1 - cur],
                        gsems[1 - cur])
                if with_count:
                    pltpu.sync_copy(ones, cacc.at[idst.at[b]], add=True)
                pltpu.sync_copy(rbufs[cur], acc.at[idst.at[b]], add=True)

        plsc.subcore_barrier()

        pltpu.sync_copy(acc.at[pl.ds(s * stripe, stripe)],
                        s_hbm.at[c, pl.ds(s * stripe, stripe)])
        if with_count:
            pltpu.sync_copy(cacc.at[pl.ds(s * stripe, stripe)],
                            cnt_hbm.at[c, pl.ds(s * stripe, stripe)])

    return k(tab, src2d, dst2d)


def _tc_mid(s1, cnt, xr, b1, w2cat):
    n = xr.shape[0]
    bn = 1000

    def body(s1_ref, cnt_ref, xr_ref, b1_ref, w_ref, z2_ref, hr_ref):
        sb = s1_ref[...]
        cb = cnt_ref[...]
        ssum = sb[0] + sb[1]
        csum = cb[0, :, 0:1] + cb[1, :, 0:1]
        mean = ssum / jnp.maximum(csum, 1.0)
        h = jnp.maximum(mean + xr_ref[...] + b1_ref[...], 0.0)
        z = lax.dot_general(h, w_ref[...], (((1,), (1,)), ((), ())),
                            preferred_element_type=jnp.float32)
        z2_ref[...] = z[:, :LANES]
        hr_ref[...] = z[:, LANES:LANES + 2]

    return pl.pallas_call(
        body,
        grid=(n // bn,),
        in_specs=[pl.BlockSpec((NC, bn, 64), lambda i: (0, i, 0)),
                  pl.BlockSpec((NC, bn, LANES), lambda i: (0, i, 0)),
                  pl.BlockSpec((bn, 64), lambda i: (i, 0)),
                  pl.BlockSpec((1, 64), lambda i: (0, 0)),
                  pl.BlockSpec((2 * LANES, 64), lambda i: (0, 0))],
        out_specs=[pl.BlockSpec((bn, LANES), lambda i: (i, 0)),
                   pl.BlockSpec((bn, 2), lambda i: (i, 0))],
        out_shape=[jax.ShapeDtypeStruct((n, LANES), jnp.float32),
                   jax.ShapeDtypeStruct((n, 2), jnp.float32)],
    )(s1, cnt, xr, b1, w2cat)


def _tc_out(s2, cnt, hr, b2):
    n = hr.shape[0]
    bn = 1000

    def body(s2_ref, cnt_ref, hr_ref, b2_ref, o_ref):
        sb = s2_ref[...]
        cb = cnt_ref[...]
        ssum = sb[0] + sb[1]
        csum = cb[0, :, 0:1] + cb[1, :, 0:1]
        v = ssum[:, 0:2] / jnp.maximum(csum, 1.0) + hr_ref[...] + b2_ref[...]
        m = jnp.max(v, axis=1, keepdims=True)
        lse = jnp.log(jnp.sum(jnp.exp(v - m), axis=1, keepdims=True))
        o_ref[...] = v - m - lse

    return pl.pallas_call(
        body,
        grid=(n // bn,),
        in_specs=[pl.BlockSpec((NC, bn, LANES), lambda i: (0, i, 0)),
                  pl.BlockSpec((NC, bn, LANES), lambda i: (0, i, 0)),
                  pl.BlockSpec((bn, 2), lambda i: (i, 0)),
                  pl.BlockSpec((1, 2), lambda i: (0, 0))],
        out_specs=pl.BlockSpec((bn, 2), lambda i: (i, 0)),
        out_shape=jax.ShapeDtypeStruct((n, 2), jnp.float32),
    )(s2, cnt, hr, b2)


def kernel(x, edge_index, W1l, W1r, b1, W2l, W2r, b2):
    n = x.shape[0]
    e = edge_index.shape[1]

    src = edge_index[0].astype(jnp.int32)
    dst = edge_index[1].astype(jnp.int32)

    block = NW * CHUNK * NBUF
    e_pad = ((e + block - 1) // block) * block
    pad = e_pad - e
    src = jnp.concatenate([src, jnp.zeros((pad,), jnp.int32)])
    dst = jnp.concatenate([dst, jnp.full((pad,), n, jnp.int32)])
    src2d = src.reshape(e_pad // CHUNK, CHUNK)
    dst2d = dst.reshape(e_pad // CHUNK, CHUNK)

    wcat = jnp.concatenate([W1l, W1r], axis=0)
    w2cat = jnp.zeros((2 * LANES, 64), jnp.float32)
    w2cat = w2cat.at[0:2].set(W2l).at[LANES:LANES + 2].set(W2r)
    b1r = b1.reshape(1, 64)
    b2r = b2.reshape(1, 2)

    y1, xr = _tc_in_proj(x, wcat)
    s1, cnt = _sc_aggregate(y1, src2d, dst2d, with_count=True, rpw0=112)
    z2, hr = _tc_mid(s1, cnt, xr, b1r, w2cat)
    (s2,) = _sc_aggregate(z2, src2d, dst2d, with_count=False, rpw0=96)
    return _tc_out(s2, cnt, hr, b2r)


# speedup vs baseline: 1.2581x; 1.0477x over previous
import functools

import jax
import jax.numpy as jnp
from jax import lax
from jax.experimental import pallas as pl
from jax.experimental.pallas import tpu as pltpu
from jax.experimental.pallas import tpu_sc as plsc

NC = 2
NS = 16
NW = NC * NS
CHUNK = 128
NBUF = 8
NRB = 2
LANES = 16


def _tc_in_proj(x, wcat):
    n, d_in = x.shape
    bn = 1000

    def body(x_ref, w_ref, y1_ref, xr_ref):
        z = lax.dot_general(x_ref[...], w_ref[...], (((1,), (1,)), ((), ())),
                            preferred_element_type=jnp.float32)
        y1_ref[...] = z[:, :64]
        xr_ref[...] = z[:, 64:]

    return pl.pallas_call(
        body,
        grid=(n // bn,),
        in_specs=[pl.BlockSpec((bn, d_in), lambda i: (i, 0)),
                  pl.BlockSpec((d_in, d_in), lambda i: (0, 0))],
        out_specs=[pl.BlockSpec((bn, 64), lambda i: (i, 0)),
                   pl.BlockSpec((bn, 64), lambda i: (i, 0))],
        out_shape=[jax.ShapeDtypeStruct((n, 64), jnp.float32),
                   jax.ShapeDtypeStruct((n, 64), jnp.float32)],
    )(x, wcat)


def _sc_aggregate(tab, src2d, dst2d, with_count, rpw0):
    n, d = tab.shape
    rtot = src2d.shape[0]
    rpw1 = rtot // NS - rpw0
    npad = ((n + 1 + NS * CHUNK - 1) // (NS * CHUNK)) * (NS * CHUNK)
    stripe = npad // NS
    zrows = npad // NS

    mesh = plsc.VectorSubcoreMesh(core_axis_name="c", subcore_axis_name="s")

    out_type = [jax.ShapeDtypeStruct((NC, npad, d), jnp.float32)]
    scratch = (
        [pltpu.VMEM((NBUF, CHUNK), jnp.int32),
         pltpu.VMEM((NBUF, CHUNK), jnp.int32)]
        + [pltpu.VMEM((CHUNK, d), jnp.float32) for _ in range(NRB)]
        + [pltpu.VMEM((CHUNK, LANES), jnp.float32),
           pltpu.VMEM_SHARED((npad, d), jnp.float32)]
        + [pltpu.SemaphoreType.DMA for _ in range(NRB)]
    )
    if with_count:
        out_type.append(jax.ShapeDtypeStruct((NC, npad, LANES), jnp.float32))
        scratch.append(pltpu.VMEM_SHARED((npad, LANES), jnp.float32))

    @functools.partial(
        pl.kernel, out_type=out_type, mesh=mesh, scratch_types=scratch,
        compiler_params=pltpu.CompilerParams(use_tc_tiling_on_sc=False))
    def k(*refs):
        nout = 2 if with_count else 1
        tab_hbm, src_hbm, dst_hbm = refs[:3]
        s_hbm = refs[3]
        cnt_hbm = refs[4] if with_count else None
        r0 = 3 + nout
        isrc, idst = refs[r0], refs[r0 + 1]
        rbufs = refs[r0 + 2:r0 + 2 + NRB]
        ones = refs[r0 + 2 + NRB]
        acc = refs[r0 + 3 + NRB]
        gsems = refs[r0 + 4 + NRB:r0 + 4 + 2 * NRB]
        cacc = refs[r0 + 4 + 2 * NRB] if with_count else None
        c = lax.axis_index("c")
        s = lax.axis_index("s")
        rpw_me = jnp.where(c == 0, rpw0, rpw1)
        base = jnp.where(c == 0, s * rpw0, NS * rpw0 + s * rpw1)

        zbuf = rbufs[0]

        @pl.loop(0, CHUNK)
        def _(r):
            ones[r, pl.ds(0, LANES)] = jnp.zeros((LANES,), jnp.float32)

            @pl.loop(0, d, step=LANES)
            def _(cc):
                zbuf[r, pl.ds(cc, LANES)] = jnp.zeros((LANES,), jnp.float32)

        @pl.loop(0, zrows, step=CHUNK)
        def _(r):
            pltpu.sync_copy(zbuf, acc.at[pl.ds(s * zrows + r, CHUNK)])
            if with_count:
                pltpu.sync_copy(ones, cacc.at[pl.ds(s * zrows + r, CHUNK)])

        if with_count:
            @pl.loop(0, CHUNK)
            def _(r):
                ones[r, pl.ds(0, LANES)] = jnp.ones((LANES,), jnp.float32)

        plsc.subcore_barrier()

        @pl.loop(0, rpw_me, step=NBUF)
        def _(m):
            r = base + m
            pltpu.sync_copy(src_hbm.at[pl.ds(r, NBUF)], isrc)
            pltpu.sync_copy(dst_hbm.at[pl.ds(r, NBUF)], idst)
            cps = [None, None]
            cps[0] = pltpu.async_copy(tab_hbm.at[isrc.at[0]], rbufs[0],
                                      gsems[0])
            for b in range(NBUF):
                cur = b % 2
                cps[cur].wait()
                if b + 1 < NBUF:
                    cps[1 - cur] = pltpu.async_copy(
                        tab_hbm.at[isrc.at[b + 1]], rbufs[1 - cur],
                        gsems[1 - cur])
                if with_count:
                    pltpu.sync_copy(ones, cacc.at[idst.at[b]], add=True)
                pltpu.sync_copy(rbufs[cur], acc.at[idst.at[b]], add=True)

        plsc.subcore_barrier()

        pltpu.sync_copy(acc.at[pl.ds(s * stripe, stripe)],
                        s_hbm.at[c, pl.ds(s * stripe, stripe)])
        if with_count:
            pltpu.sync_copy(cacc.at[pl.ds(s * stripe, stripe)],
                            cnt_hbm.at[c, pl.ds(s * stripe, stripe)])

    return k(tab, src2d, dst2d)


def _tc_mid(s1, cnt, xr, b1, w2cat):
    n = xr.shape[0]
    bn = 1000

    def body(s1_ref, cnt_ref, xr_ref, b1_ref, w_ref, z2_ref, hr_ref):
        sb = s1_ref[...]
        cb = cnt_ref[...]
        ssum = sb[0] + sb[1]
        csum = cb[0, :, 0:1] + cb[1, :, 0:1]
        mean = ssum / jnp.maximum(csum, 1.0)
        h = jnp.maximum(mean + xr_ref[...] + b1_ref[...], 0.0)
        z = lax.dot_general(h, w_ref[...], (((1,), (1,)), ((), ())),
                            preferred_element_type=jnp.float32)
        z2_ref[...] = z[:, :LANES]
        hr_ref[...] = z[:, LANES:LANES + 2]

    return pl.pallas_call(
        body,
        grid=(n // bn,),
        in_specs=[pl.BlockSpec((NC, bn, 64), lambda i: (0, i, 0)),
                  pl.BlockSpec((NC, bn, LANES), lambda i: (0, i, 0)),
                  pl.BlockSpec((bn, 64), lambda i: (i, 0)),
                  pl.BlockSpec((1, 64), lambda i: (0, 0)),
                  pl.BlockSpec((2 * LANES, 64), lambda i: (0, 0))],
        out_specs=[pl.BlockSpec((bn, LANES), lambda i: (i, 0)),
                   pl.BlockSpec((bn, 2), lambda i: (i, 0))],
        out_shape=[jax.ShapeDtypeStruct((n, LANES), jnp.float32),
                   jax.ShapeDtypeStruct((n, 2), jnp.float32)],
    )(s1, cnt, xr, b1, w2cat)


def _tc_out(s2, cnt, hr, b2):
    n = hr.shape[0]
    bn = 1000

    def body(s2_ref, cnt_ref, hr_ref, b2_ref, o_ref):
        sb = s2_ref[...]
        cb = cnt_ref[...]
        ssum = sb[0] + sb[1]
        csum = cb[0, :, 0:1] + cb[1, :, 0:1]
        v = ssum[:, 0:2] / jnp.maximum(csum, 1.0) + hr_ref[...] + b2_ref[...]
        m = jnp.max(v, axis=1, keepdims=True)
        lse = jnp.log(jnp.sum(jnp.exp(v - m), axis=1, keepdims=True))
        o_ref[...] = v - m - lse

    return pl.pallas_call(
        body,
        grid=(n // bn,),
        in_specs=[pl.BlockSpec((NC, bn, LANES), lambda i: (0, i, 0)),
                  pl.BlockSpec((NC, bn, LANES), lambda i: (0, i, 0)),
                  pl.BlockSpec((bn, 2), lambda i: (i, 0)),
                  pl.BlockSpec((1, 2), lambda i: (0, 0))],
        out_specs=pl.BlockSpec((bn, 2), lambda i: (i, 0)),
        out_shape=jax.ShapeDtypeStruct((n, 2), jnp.float32),
    )(s2, cnt, hr, b2)


def kernel(x, edge_index, W1l, W1r, b1, W2l, W2r, b2):
    n = x.shape[0]
    e = edge_index.shape[1]

    src = edge_index[0].astype(jnp.int32)
    dst = edge_index[1].astype(jnp.int32)

    block = NW * CHUNK * NBUF
    e_pad = ((e + block - 1) // block) * block
    pad = e_pad - e
    src = jnp.concatenate([src, jnp.zeros((pad,), jnp.int32)])
    dst = jnp.concatenate([dst, jnp.full((pad,), n, jnp.int32)])
    src2d = src.reshape(e_pad // CHUNK, CHUNK)
    dst2d = dst.reshape(e_pad // CHUNK, CHUNK)

    wcat = jnp.concatenate([W1l, W1r], axis=0)
    w2cat = jnp.zeros((2 * LANES, 64), jnp.float32)
    w2cat = w2cat.at[0:2].set(W2l).at[LANES:LANES + 2].set(W2r)
    b1r = b1.reshape(1, 64)
    b2r = b2.reshape(1, 2)

    y1, xr = _tc_in_proj(x, wcat)
    s1, cnt = _sc_aggregate(y1, src2d, dst2d, with_count=True, rpw0=128)
    z2, hr = _tc_mid(s1, cnt, xr, b1r, w2cat)
    (s2,) = _sc_aggregate(z2, src2d, dst2d, with_count=False, rpw0=96)
    return _tc_out(s2, cnt, hr, b2r)


# speedup vs baseline: 1.2731x; 1.0120x over previous
import functools

import jax
import jax.numpy as jnp
from jax import lax
from jax.experimental import pallas as pl
from jax.experimental.pallas import tpu as pltpu
from jax.experimental.pallas import tpu_sc as plsc

NC = 2
NS = 16
NW = NC * NS
CHUNK = 128
NBUF = 8
NRB = 2
LANES = 16


def _tc_in_proj(x, wcat):
    n, d_in = x.shape
    bn = 1000

    def body(x_ref, w_ref, y1_ref, xr_ref):
        z = lax.dot_general(x_ref[...], w_ref[...], (((1,), (1,)), ((), ())),
                            preferred_element_type=jnp.float32)
        y1_ref[...] = z[:, :64]
        xr_ref[...] = z[:, 64:]

    return pl.pallas_call(
        body,
        grid=(n // bn,),
        in_specs=[pl.BlockSpec((bn, d_in), lambda i: (i, 0)),
                  pl.BlockSpec((d_in, d_in), lambda i: (0, 0))],
        out_specs=[pl.BlockSpec((bn, 64), lambda i: (i, 0)),
                   pl.BlockSpec((bn, 64), lambda i: (i, 0))],
        out_shape=[jax.ShapeDtypeStruct((n, 64), jnp.float32),
                   jax.ShapeDtypeStruct((n, 64), jnp.float32)],
    )(x, wcat)


def _sc_aggregate(tab, src2d, dst2d, with_count, rpw0):
    n, d = tab.shape
    rtot = src2d.shape[0]
    rpw1 = rtot // NS - rpw0
    npad = ((n + 1 + NS * CHUNK - 1) // (NS * CHUNK)) * (NS * CHUNK)
    stripe = npad // NS
    zrows = npad // NS

    mesh = plsc.VectorSubcoreMesh(core_axis_name="c", subcore_axis_name="s")

    out_type = [jax.ShapeDtypeStruct((NC, npad, d), jnp.float32)]
    scratch = (
        [pltpu.VMEM((NBUF, CHUNK), jnp.int32),
         pltpu.VMEM((NBUF, CHUNK), jnp.int32)]
        + [pltpu.VMEM((CHUNK, d), jnp.float32) for _ in range(NRB)]
        + [pltpu.VMEM((CHUNK, LANES), jnp.float32),
           pltpu.VMEM_SHARED((npad, d), jnp.float32)]
        + [pltpu.SemaphoreType.DMA for _ in range(NRB)]
    )
    if with_count:
        out_type.append(jax.ShapeDtypeStruct((NC, npad, LANES), jnp.float32))
        scratch.append(pltpu.VMEM_SHARED((npad, LANES), jnp.float32))

    @functools.partial(
        pl.kernel, out_type=out_type, mesh=mesh, scratch_types=scratch,
        compiler_params=pltpu.CompilerParams(use_tc_tiling_on_sc=False))
    def k(*refs):
        nout = 2 if with_count else 1
        tab_hbm, src_hbm, dst_hbm = refs[:3]
        s_hbm = refs[3]
        cnt_hbm = refs[4] if with_count else None
        r0 = 3 + nout
        isrc, idst = refs[r0], refs[r0 + 1]
        rbufs = refs[r0 + 2:r0 + 2 + NRB]
        ones = refs[r0 + 2 + NRB]
        acc = refs[r0 + 3 + NRB]
        gsems = refs[r0 + 4 + NRB:r0 + 4 + 2 * NRB]
        cacc = refs[r0 + 4 + 2 * NRB] if with_count else None
        c = lax.axis_index("c")
        s = lax.axis_index("s")
        rpw_me = jnp.where(c == 0, rpw0, rpw1)
        base = jnp.where(c == 0, s * rpw0, NS * rpw0 + s * rpw1)

        zbuf = rbufs[0]

        @pl.loop(0, CHUNK)
        def _(r):
            ones[r, pl.ds(0, LANES)] = jnp.zeros((LANES,), jnp.float32)

            @pl.loop(0, d, step=LANES)
            def _(cc):
                zbuf[r, pl.ds(cc, LANES)] = jnp.zeros((LANES,), jnp.float32)

        @pl.loop(0, zrows, step=CHUNK)
        def _(r):
            pltpu.sync_copy(zbuf, acc.at[pl.ds(s * zrows + r, CHUNK)])
            if with_count:
                pltpu.sync_copy(ones, cacc.at[pl.ds(s * zrows + r, CHUNK)])

        if with_count:
            @pl.loop(0, CHUNK)
            def _(r):
                ones[r, pl.ds(0, LANES)] = jnp.ones((LANES,), jnp.float32)

        plsc.subcore_barrier()

        @pl.loop(0, rpw_me, step=NBUF)
        def _(m):
            r = base + m
            pltpu.sync_copy(src_hbm.at[pl.ds(r, NBUF)], isrc)
            pltpu.sync_copy(dst_hbm.at[pl.ds(r, NBUF)], idst)
            cps = [None, None]
            cps[0] = pltpu.async_copy(tab_hbm.at[isrc.at[0]], rbufs[0],
                                      gsems[0])
            for b in range(NBUF):
                cur = b % 2
                cps[cur].wait()
                if b + 1 < NBUF:
                    cps[1 - cur] = pltpu.async_copy(
                        tab_hbm.at[isrc.at[b + 1]], rbufs[1 - cur],
                        gsems[1 - cur])
                if with_count:
                    pltpu.sync_copy(ones, cacc.at[idst.at[b]], add=True)
                pltpu.sync_copy(rbufs[cur], acc.at[idst.at[b]], add=True)

        plsc.subcore_barrier()

        pltpu.sync_copy(acc.at[pl.ds(s * stripe, stripe)],
                        s_hbm.at[c, pl.ds(s * stripe, stripe)])
        if with_count:
            pltpu.sync_copy(cacc.at[pl.ds(s * stripe, stripe)],
                            cnt_hbm.at[c, pl.ds(s * stripe, stripe)])

    return k(tab, src2d, dst2d)


def _tc_mid(s1, cnt, xr, b1, w2cat):
    n = xr.shape[0]
    bn = 1000

    def body(s1_ref, cnt_ref, xr_ref, b1_ref, w_ref, z2_ref, hr_ref):
        sb = s1_ref[...]
        cb = cnt_ref[...]
        ssum = sb[0] + sb[1]
        csum = cb[0, :, 0:1] + cb[1, :, 0:1]
        mean = ssum / jnp.maximum(csum, 1.0)
        h = jnp.maximum(mean + xr_ref[...] + b1_ref[...], 0.0)
        z = lax.dot_general(h, w_ref[...], (((1,), (1,)), ((), ())),
                            preferred_element_type=jnp.float32)
        z2_ref[...] = z[:, :LANES]
        hr_ref[...] = z[:, LANES:LANES + 2]

    return pl.pallas_call(
        body,
        grid=(n // bn,),
        in_specs=[pl.BlockSpec((NC, bn, 64), lambda i: (0, i, 0)),
                  pl.BlockSpec((NC, bn, LANES), lambda i: (0, i, 0)),
                  pl.BlockSpec((bn, 64), lambda i: (i, 0)),
                  pl.BlockSpec((1, 64), lambda i: (0, 0)),
                  pl.BlockSpec((2 * LANES, 64), lambda i: (0, 0))],
        out_specs=[pl.BlockSpec((bn, LANES), lambda i: (i, 0)),
                   pl.BlockSpec((bn, 2), lambda i: (i, 0))],
        out_shape=[jax.ShapeDtypeStruct((n, LANES), jnp.float32),
                   jax.ShapeDtypeStruct((n, 2), jnp.float32)],
    )(s1, cnt, xr, b1, w2cat)


def _tc_out(s2, cnt, hr, b2):
    n = hr.shape[0]
    bn = 1000

    def body(s2_ref, cnt_ref, hr_ref, b2_ref, o_ref):
        sb = s2_ref[...]
        cb = cnt_ref[...]
        ssum = sb[0] + sb[1]
        csum = cb[0, :, 0:1] + cb[1, :, 0:1]
        v = ssum[:, 0:2] / jnp.maximum(csum, 1.0) + hr_ref[...] + b2_ref[...]
        m = jnp.max(v, axis=1, keepdims=True)
        lse = jnp.log(jnp.sum(jnp.exp(v - m), axis=1, keepdims=True))
        o_ref[...] = v - m - lse

    return pl.pallas_call(
        body,
        grid=(n // bn,),
        in_specs=[pl.BlockSpec((NC, bn, LANES), lambda i: (0, i, 0)),
                  pl.BlockSpec((NC, bn, LANES), lambda i: (0, i, 0)),
                  pl.BlockSpec((bn, 2), lambda i: (i, 0)),
                  pl.BlockSpec((1, 2), lambda i: (0, 0))],
        out_specs=pl.BlockSpec((bn, 2), lambda i: (i, 0)),
        out_shape=jax.ShapeDtypeStruct((n, 2), jnp.float32),
    )(s2, cnt, hr, b2)


def kernel(x, edge_index, W1l, W1r, b1, W2l, W2r, b2):
    n = x.shape[0]
    e = edge_index.shape[1]

    src = edge_index[0].astype(jnp.int32)
    dst = edge_index[1].astype(jnp.int32)

    block = NW * CHUNK * NBUF
    e_pad = ((e + block - 1) // block) * block
    pad = e_pad - e
    src = jnp.concatenate([src, jnp.zeros((pad,), jnp.int32)])
    dst = jnp.concatenate([dst, jnp.full((pad,), n, jnp.int32)])
    src2d = src.reshape(e_pad // CHUNK, CHUNK)
    dst2d = dst.reshape(e_pad // CHUNK, CHUNK)

    wcat = jnp.concatenate([W1l, W1r], axis=0)
    w2cat = jnp.zeros((2 * LANES, 64), jnp.float32)
    w2cat = w2cat.at[0:2].set(W2l).at[LANES:LANES + 2].set(W2r)
    b1r = b1.reshape(1, 64)
    b2r = b2.reshape(1, 2)

    y1, xr = _tc_in_proj(x, wcat)
    s1, cnt = _sc_aggregate(y1, src2d, dst2d, with_count=True, rpw0=136)
    z2, hr = _tc_mid(s1, cnt, xr, b1r, w2cat)
    (s2,) = _sc_aggregate(z2, src2d, dst2d, with_count=False, rpw0=104)
    return _tc_out(s2, cnt, hr, b2r)
